# baseline (speedup 1.0000x reference)
import jax
import jax.numpy as jnp
from jax import lax
from jax.experimental import pallas as pl
from jax.experimental.pallas import tpu as pltpu
from jax.experimental.pallas import tpu_sc as plsc

_B, _C, _H, _W = 4, 32, 64, 2048
_N = _H * _W
_NX = _NY = 512
_NCELL = _NX * _NY
_NCHUNKS = 16
_CHUNK = _NCELL // _NCHUNKS
_DUMPS = 16
_ROWS = _CHUNK + _DUMPS

_NCORES = 2
_NSUB = 16
_PTS_PER_TILE = _N // _NSUB
_SB = 1024
_NSB = _PTS_PER_TILE // _SB
_G = 128
_NG = _SB // _G

_TROWS = _CHUNK // _NSUB
_YROWS = _TROWS // _NX
_ZR = 256
_MS = 512

_XY0 = -51.2
_VOX = 0.2


def _sc_body(feats_hbm, pim_hbm, ms_hbm, canv_hbm,
             feats_v, gidx_v, x_v, y_v, m_v, idx_v, ones_v, zrow_v, zcnt_v,
             mrow_v, mcnt_v, rcp_v, mout_v, sums_sh, cnts_sh):
  cid = lax.axis_index("c")
  tid = lax.axis_index("s")
  lanes = jnp.arange(16, dtype=jnp.int32)
  ones16 = jnp.ones((16,), jnp.float32)
  zeros16 = jnp.zeros((16,), jnp.float32)

  for g in range(_G // 16):
    ones_v[pl.ds(g * 16, 16)] = ones16

  def _zr(i, c):
    zrow_v[i, pl.ds(0, 16)] = zeros16
    zrow_v[i, pl.ds(16, 16)] = zeros16
    return c
  lax.fori_loop(0, _ZR, _zr, 0)

  def _zc(i, c):
    zcnt_v[pl.ds(i * 16, 16)] = zeros16
    return c
  lax.fori_loop(0, _SB // 16, _zc, 0)

  def _batch(b, carry0):
    for sb in range(_NSB):
      pbase = tid * _PTS_PER_TILE + sb * _SB
      pltpu.sync_copy(pim_hbm.at[b, 0, pl.ds(pbase, _SB)], x_v)
      pltpu.sync_copy(pim_hbm.at[b, 1, pl.ds(pbase, _SB)], y_v)
      pltpu.sync_copy(ms_hbm.at[b, pl.ds(pbase, _SB)], m_v)

      def _ci(k, c, sb=sb):
        o = k * 16
        xx = x_v[pl.ds(o, 16)]
        yy = y_v[pl.ds(o, 16)]
        mm = m_v[pl.ds(o, 16)]
        cx = ((xx - _XY0) / _VOX).astype(jnp.int32)
        cx = jnp.minimum(jnp.maximum(cx, 0), _NX - 1)
        cy = ((yy - _XY0) / _VOX).astype(jnp.int32)
        cy = jnp.minimum(jnp.maximum(cy, 0), _NY - 1)
        flat = cy * _NX + cx
        flat = jnp.where(mm > 0, flat, _NCELL)
        gidx_v[pl.ds(sb * _SB + o, 16)] = flat
        return c
      lax.fori_loop(0, _SB // 16, _ci, 0)

    def _pass(i, carry1):
      ch = i * _NCORES + cid
      cell0 = ch * _CHUNK

      for r in range(_TROWS // _ZR):
        pltpu.sync_copy(zrow_v, sums_sh.at[pl.ds(tid * _TROWS + r * _ZR, _ZR)])
      for r in range(_TROWS // _SB):
        pltpu.sync_copy(zcnt_v, cnts_sh.at[pl.ds(tid * _TROWS + r * _SB, _SB)])

      @pl.when(tid == 0)
      def _():
        pltpu.sync_copy(zrow_v.at[pl.ds(0, _DUMPS)],
                        sums_sh.at[pl.ds(_CHUNK, _DUMPS)])
        pltpu.sync_copy(zcnt_v.at[pl.ds(0, _DUMPS)],
                        cnts_sh.at[pl.ds(_CHUNK, _DUMPS)])

      plsc.subcore_barrier()

      for sb in range(_NSB):
        pbase = tid * _PTS_PER_TILE + sb * _SB
        pltpu.sync_copy(feats_hbm.at[b, pl.ds(pbase, _SB)], feats_v)

        for g in range(_NG):
          def _li(j, c, g=g, sb=sb):
            o = sb * _SB + g * _G + j * 16
            fl = gidx_v[pl.ds(o, 16)]
            loc = fl - cell0
            ok = (loc >= 0) & (loc < _CHUNK)
            idx_v[g, pl.ds(j * 16, 16)] = jnp.where(ok, loc, _CHUNK + lanes)
            return c
          lax.fori_loop(0, _G // 16, _li, 0)

        for g in range(_NG):
          pltpu.sync_copy(feats_v.at[pl.ds(g * _G, _G)],
                          sums_sh.at[idx_v.at[g]], add=True)
          pltpu.sync_copy(ones_v, cnts_sh.at[idx_v.at[g]], add=True)

      plsc.subcore_barrier()

      for s in range(_TROWS // _MS):
        row0 = tid * _TROWS + s * _MS
        pltpu.sync_copy(sums_sh.at[pl.ds(row0, _MS)], mrow_v)
        pltpu.sync_copy(cnts_sh.at[pl.ds(row0, _MS)], mcnt_v)

        def _rcp(k, c):
          cc = mcnt_v[pl.ds(k * 16, 16)]
          rcp_v[pl.ds(k * 16, 16)] = 1.0 / jnp.maximum(cc, 1.0)
          return c
        lax.fori_loop(0, _MS // 16, _rcp, 0)

        def _tc(c, carry2):
          cvec = jnp.full((16,), c, jnp.int32)

          def _tg(g, c2, cvec=cvec):
            cells = g * 16 + lanes
            vals = plsc.load_gather(mrow_v, [cells, cvec])
            mout_v[c, pl.ds(g * 16, 16)] = vals * rcp_v[pl.ds(g * 16, 16)]
            return c2
          lax.fori_loop(0, _MS // 16, _tg, 0)
          return carry2
        lax.fori_loop(0, _C, _tc, 0)

        yrow = ch * (_CHUNK // _NX) + tid * _YROWS + s
        for c in range(_C):
          pltpu.sync_copy(mout_v.at[c], canv_hbm.at[b, c, yrow])

      plsc.subcore_barrier()
      return carry1

    lax.fori_loop(0, _NCHUNKS // _NCORES, _pass, 0)
    return carry0

  lax.fori_loop(0, _B, _batch, 0)


def _sc_scatter(feats_t, pim, ms):
  mesh = plsc.VectorSubcoreMesh(core_axis_name="c", subcore_axis_name="s",
                                num_cores=_NCORES, num_subcores=_NSUB)
  return pl.kernel(
      _sc_body,
      out_type=jax.ShapeDtypeStruct((_B, _C, _NY, _NX), jnp.float32),
      mesh=mesh,
      compiler_params=pltpu.CompilerParams(use_tc_tiling_on_sc=False,
                                           needs_layout_passes=False),
      scratch_types=[
          pltpu.VMEM((_SB, _C), jnp.float32),
          pltpu.VMEM((_PTS_PER_TILE,), jnp.int32),
          pltpu.VMEM((_SB,), jnp.float32),
          pltpu.VMEM((_SB,), jnp.float32),
          pltpu.VMEM((_SB,), jnp.int32),
          pltpu.VMEM((_NG, _G), jnp.int32),
          pltpu.VMEM((_G,), jnp.float32),
          pltpu.VMEM((_ZR, _C), jnp.float32),
          pltpu.VMEM((_SB,), jnp.float32),
          pltpu.VMEM((_MS, _C), jnp.float32),
          pltpu.VMEM((_MS,), jnp.float32),
          pltpu.VMEM((_MS,), jnp.float32),
          pltpu.VMEM((_C, _MS), jnp.float32),
          pltpu.VMEM_SHARED((_ROWS, _C), jnp.float32),
          pltpu.VMEM_SHARED((_ROWS,), jnp.float32),
      ],
  )(feats_t, pim, ms)


def kernel(fv_features, points_img, proj_masks):
  feats_t = jnp.transpose(fv_features.reshape(_B, _C, _N), (0, 2, 1))
  pim = points_img.reshape(_B, 4, _N)
  ms = proj_masks.reshape(_B, _N)
  return _sc_scatter(feats_t, pim, ms)


# speedup vs baseline: 1.1912x; 1.1912x over previous
import jax
import jax.numpy as jnp
from jax import lax
from jax.experimental import pallas as pl
from jax.experimental.pallas import tpu as pltpu
from jax.experimental.pallas import tpu_sc as plsc

_B, _C, _H, _W = 4, 32, 64, 2048
_N = _H * _W
_NX = _NY = 512
_NCELL = _NX * _NY
_NCHUNKS = 16
_CHUNK = _NCELL // _NCHUNKS
_DUMPS = 16
_ROWS = _CHUNK + _DUMPS

_NCORES = 2
_NSUB = 16
_PTS_PER_TILE = _N // _NSUB
_SB = 1024
_NSB = _PTS_PER_TILE // _SB
_G = 128
_NG = _SB // _G

_TROWS = _CHUNK // _NSUB
_YROWS = _TROWS // _NX
_ZR = 256
_MS = 512

_XY0 = -51.2
_VOX = 0.2


def _sc_body(feats_hbm, pim_hbm, ms_hbm, mean_hbm,
             feats_v, gidx_v, x_v, y_v, m_v, idx_v, ones_v, zrow_v, zcnt_v,
             mrow_v, mcnt_v, rcp_v, sums_sh, cnts_sh):
  cid = lax.axis_index("c")
  tid = lax.axis_index("s")
  lanes = jnp.arange(16, dtype=jnp.int32)
  ones16 = jnp.ones((16,), jnp.float32)
  zeros16 = jnp.zeros((16,), jnp.float32)

  for g in range(_G // 16):
    ones_v[pl.ds(g * 16, 16)] = ones16

  def _zr(i, c):
    zrow_v[i, pl.ds(0, 16)] = zeros16
    zrow_v[i, pl.ds(16, 16)] = zeros16
    return c
  lax.fori_loop(0, _ZR, _zr, 0)

  def _zc(i, c):
    zcnt_v[pl.ds(i * 16, 16)] = zeros16
    return c
  lax.fori_loop(0, _SB // 16, _zc, 0)

  def _batch(b, carry0):
    for sb in range(_NSB):
      pbase = tid * _PTS_PER_TILE + sb * _SB
      pltpu.sync_copy(pim_hbm.at[b, 0, pl.ds(pbase, _SB)], x_v)
      pltpu.sync_copy(pim_hbm.at[b, 1, pl.ds(pbase, _SB)], y_v)
      pltpu.sync_copy(ms_hbm.at[b, pl.ds(pbase, _SB)], m_v)

      def _ci(k, c, sb=sb):
        o = k * 16
        xx = x_v[pl.ds(o, 16)]
        yy = y_v[pl.ds(o, 16)]
        mm = m_v[pl.ds(o, 16)]
        cx = ((xx - _XY0) / _VOX).astype(jnp.int32)
        cx = jnp.minimum(jnp.maximum(cx, 0), _NX - 1)
        cy = ((yy - _XY0) / _VOX).astype(jnp.int32)
        cy = jnp.minimum(jnp.maximum(cy, 0), _NY - 1)
        flat = cy * _NX + cx
        flat = jnp.where(mm > 0, flat, _NCELL)
        gidx_v[pl.ds(sb * _SB + o, 16)] = flat
        return c
      lax.fori_loop(0, _SB // 16, _ci, 0)

    def _pass(i, carry1):
      ch = i * _NCORES + cid
      cell0 = ch * _CHUNK

      for r in range(_TROWS // _ZR):
        pltpu.sync_copy(zrow_v, sums_sh.at[pl.ds(tid * _TROWS + r * _ZR, _ZR)])
      for r in range(_TROWS // _SB):
        pltpu.sync_copy(zcnt_v, cnts_sh.at[pl.ds(tid * _TROWS + r * _SB, _SB)])

      @pl.when(tid == 0)
      def _():
        pltpu.sync_copy(zrow_v.at[pl.ds(0, _DUMPS)],
                        sums_sh.at[pl.ds(_CHUNK, _DUMPS)])
        pltpu.sync_copy(zcnt_v.at[pl.ds(0, _DUMPS)],
                        cnts_sh.at[pl.ds(_CHUNK, _DUMPS)])

      plsc.subcore_barrier()

      for sb in range(_NSB):
        pbase = tid * _PTS_PER_TILE + sb * _SB
        pltpu.sync_copy(feats_hbm.at[b, pl.ds(pbase, _SB)], feats_v)

        for g in range(_NG):
          def _li(j, c, g=g, sb=sb):
            o = sb * _SB + g * _G + j * 16
            fl = gidx_v[pl.ds(o, 16)]
            loc = fl - cell0
            ok = (loc >= 0) & (loc < _CHUNK)
            idx_v[g, pl.ds(j * 16, 16)] = jnp.where(ok, loc, _CHUNK + lanes)
            return c
          lax.fori_loop(0, _G // 16, _li, 0)

        for g in range(_NG):
          pltpu.sync_copy(feats_v.at[pl.ds(g * _G, _G)],
                          sums_sh.at[idx_v.at[g]], add=True)
          pltpu.sync_copy(ones_v, cnts_sh.at[idx_v.at[g]], add=True)

      plsc.subcore_barrier()

      for s in range(_TROWS // _MS):
        row0 = tid * _TROWS + s * _MS
        pltpu.sync_copy(sums_sh.at[pl.ds(row0, _MS)], mrow_v)
        pltpu.sync_copy(cnts_sh.at[pl.ds(row0, _MS)], mcnt_v)

        def _rcp(k, c):
          cc = mcnt_v[pl.ds(k * 16, 16)]
          rcp_v[pl.ds(k * 16, 16)] = 1.0 / jnp.maximum(cc, 1.0)
          return c
        lax.fori_loop(0, _MS // 16, _rcp, 0)

        def _mm(j, c):
          r = rcp_v[pl.ds(j, 16)][0]
          mrow_v[j, pl.ds(0, 16)] = mrow_v[j, pl.ds(0, 16)] * r
          mrow_v[j, pl.ds(16, 16)] = mrow_v[j, pl.ds(16, 16)] * r
          return c
        lax.fori_loop(0, _MS, _mm, 0)

        pltpu.sync_copy(mrow_v,
                        mean_hbm.at[b, pl.ds(cell0 + row0, _MS)])

      plsc.subcore_barrier()
      return carry1

    lax.fori_loop(0, _NCHUNKS // _NCORES, _pass, 0)
    return carry0

  lax.fori_loop(0, _B, _batch, 0)


def _sc_scatter(feats_t, pim, ms):
  mesh = plsc.VectorSubcoreMesh(core_axis_name="c", subcore_axis_name="s",
                                num_cores=_NCORES, num_subcores=_NSUB)
  return pl.kernel(
      _sc_body,
      out_type=jax.ShapeDtypeStruct((_B, _NCELL, _C), jnp.float32),
      mesh=mesh,
      compiler_params=pltpu.CompilerParams(use_tc_tiling_on_sc=False,
                                           needs_layout_passes=False),
      scratch_types=[
          pltpu.VMEM((_SB, _C), jnp.float32),
          pltpu.VMEM((_PTS_PER_TILE,), jnp.int32),
          pltpu.VMEM((_SB,), jnp.float32),
          pltpu.VMEM((_SB,), jnp.float32),
          pltpu.VMEM((_SB,), jnp.int32),
          pltpu.VMEM((_NG, _G), jnp.int32),
          pltpu.VMEM((_G,), jnp.float32),
          pltpu.VMEM((_ZR, _C), jnp.float32),
          pltpu.VMEM((_SB,), jnp.float32),
          pltpu.VMEM((_MS, _C), jnp.float32),
          pltpu.VMEM((_MS,), jnp.float32),
          pltpu.VMEM((_MS + 16,), jnp.float32),
          pltpu.VMEM_SHARED((_ROWS, _C), jnp.float32),
          pltpu.VMEM_SHARED((_ROWS,), jnp.float32),
      ],
  )(feats_t, pim, ms)


def kernel(fv_features, points_img, proj_masks):
  feats_t = jnp.transpose(fv_features.reshape(_B, _C, _N), (0, 2, 1))
  pim = points_img.reshape(_B, 4, _N)
  ms = proj_masks.reshape(_B, _N)
  mean = _sc_scatter(feats_t, pim, ms)
  return jnp.transpose(mean, (0, 2, 1)).reshape(_B, _C, _NY, _NX)


# speedup vs baseline: 1.2686x; 1.0650x over previous
import jax
import jax.numpy as jnp
from jax import lax
from jax.experimental import pallas as pl
from jax.experimental.pallas import tpu as pltpu
from jax.experimental.pallas import tpu_sc as plsc

_B, _C, _H, _W = 4, 32, 64, 2048
_N = _H * _W
_NX = _NY = 512
_NCELL = _NX * _NY
_NCHUNKS = 16
_CHUNK = _NCELL // _NCHUNKS
_DUMPS = 16
_ROWS = _CHUNK + _DUMPS

_NCORES = 2
_NSUB = 16
_PTS_PER_TILE = _N // _NSUB
_SB = 1024
_NSB = _PTS_PER_TILE // _SB
_G = 128
_NG = _SB // _G

_TROWS = _CHUNK // _NSUB
_YROWS = _TROWS // _NX
_ZR = 256
_MS = 512

_XY0 = -51.2
_VOX = 0.2


def _sc_body(feats_hbm, pim_hbm, ms_hbm, mean_hbm,
             feats_v, gidx_v, x_v, y_v, m_v, idx_v, ones_v, zrow_v, zcnt_v,
             mrow_v, mcnt_v, rcp_v, sums_sh, cnts_sh):
  cid = lax.axis_index("c")
  tid = lax.axis_index("s")
  lanes = jnp.arange(16, dtype=jnp.int32)
  ones16 = jnp.ones((16,), jnp.float32)
  zeros16 = jnp.zeros((16,), jnp.float32)

  for g in range(_G // 16):
    ones_v[pl.ds(g * 16, 16)] = ones16

  def _zr(i, c):
    zrow_v[i, pl.ds(0, 16)] = zeros16
    zrow_v[i, pl.ds(16, 16)] = zeros16
    return c
  lax.fori_loop(0, _ZR, _zr, 0)

  def _zc(i, c):
    zcnt_v[pl.ds(i * 16, 16)] = zeros16
    return c
  lax.fori_loop(0, _SB // 16, _zc, 0)

  def _batch(b, carry0):
    for sb in range(_NSB):
      pbase = tid * _PTS_PER_TILE + sb * _SB
      pltpu.sync_copy(pim_hbm.at[b, 0, pl.ds(pbase, _SB)], x_v)
      pltpu.sync_copy(pim_hbm.at[b, 1, pl.ds(pbase, _SB)], y_v)
      pltpu.sync_copy(ms_hbm.at[b, pl.ds(pbase, _SB)], m_v)

      def _ci(k, sb=sb):
        o = k * 16
        xx = x_v[pl.ds(o, 16)]
        yy = y_v[pl.ds(o, 16)]
        mm = m_v[pl.ds(o, 16)]
        cx = ((xx - _XY0) / _VOX).astype(jnp.int32)
        cx = jnp.minimum(jnp.maximum(cx, 0), _NX - 1)
        cy = ((yy - _XY0) / _VOX).astype(jnp.int32)
        cy = jnp.minimum(jnp.maximum(cy, 0), _NY - 1)
        flat = cy * _NX + cx
        flat = jnp.where(mm > 0, flat, _NCELL)
        gidx_v[pl.ds(sb * _SB + o, 16)] = flat
      plsc.parallel_loop(0, _SB // 16, 1, unroll=4)(_ci)

    def _pass(i, carry1):
      ch = i * _NCORES + cid
      cell0 = ch * _CHUNK

      for r in range(_TROWS // _ZR):
        pltpu.sync_copy(zrow_v, sums_sh.at[pl.ds(tid * _TROWS + r * _ZR, _ZR)])
      for r in range(_TROWS // _SB):
        pltpu.sync_copy(zcnt_v, cnts_sh.at[pl.ds(tid * _TROWS + r * _SB, _SB)])

      @pl.when(tid == 0)
      def _():
        pltpu.sync_copy(zrow_v.at[pl.ds(0, _DUMPS)],
                        sums_sh.at[pl.ds(_CHUNK, _DUMPS)])
        pltpu.sync_copy(zcnt_v.at[pl.ds(0, _DUMPS)],
                        cnts_sh.at[pl.ds(_CHUNK, _DUMPS)])

      plsc.subcore_barrier()

      for sb in range(_NSB):
        pbase = tid * _PTS_PER_TILE + sb * _SB
        pltpu.sync_copy(feats_hbm.at[b, pl.ds(pbase, _SB)], feats_v)

        def _li(k, sb=sb):
          fl = gidx_v[pl.ds(sb * _SB + k * 16, 16)]
          loc = fl - cell0
          ok = (loc >= 0) & (loc < _CHUNK)
          idx_v[k // 8, pl.ds((k % 8) * 16, 16)] = jnp.where(
              ok, loc, _CHUNK + lanes)
        plsc.parallel_loop(0, _SB // 16, 1, unroll=4)(_li)

        for g in range(_NG):
          pltpu.sync_copy(feats_v.at[pl.ds(g * _G, _G)],
                          sums_sh.at[idx_v.at[g]], add=True)
          pltpu.sync_copy(ones_v, cnts_sh.at[idx_v.at[g]], add=True)

      plsc.subcore_barrier()

      for s in range(_TROWS // _MS):
        row0 = tid * _TROWS + s * _MS
        pltpu.sync_copy(sums_sh.at[pl.ds(row0, _MS)], mrow_v)
        pltpu.sync_copy(cnts_sh.at[pl.ds(row0, _MS)], mcnt_v)

        def _rcp(k):
          cc = mcnt_v[pl.ds(k * 16, 16)]
          rcp_v[pl.ds(k * 16, 16)] = 1.0 / jnp.maximum(cc, 1.0)
        plsc.parallel_loop(0, _MS // 16, 1, unroll=2)(_rcp)

        def _mm(j):
          r = rcp_v[pl.ds(j, 16)][0]
          mrow_v[j, pl.ds(0, 16)] = mrow_v[j, pl.ds(0, 16)] * r
          mrow_v[j, pl.ds(16, 16)] = mrow_v[j, pl.ds(16, 16)] * r
        plsc.parallel_loop(0, _MS, 1, unroll=8)(_mm)

        pltpu.sync_copy(mrow_v,
                        mean_hbm.at[b, pl.ds(cell0 + row0, _MS)])

      plsc.subcore_barrier()
      return carry1

    lax.fori_loop(0, _NCHUNKS // _NCORES, _pass, 0)
    return carry0

  lax.fori_loop(0, _B, _batch, 0)


def _sc_scatter(feats_t, pim, ms):
  mesh = plsc.VectorSubcoreMesh(core_axis_name="c", subcore_axis_name="s",
                                num_cores=_NCORES, num_subcores=_NSUB)
  return pl.kernel(
      _sc_body,
      out_type=jax.ShapeDtypeStruct((_B, _NCELL, _C), jnp.float32),
      mesh=mesh,
      compiler_params=pltpu.CompilerParams(use_tc_tiling_on_sc=False,
                                           needs_layout_passes=False),
      scratch_types=[
          pltpu.VMEM((_SB, _C), jnp.float32),
          pltpu.VMEM((_PTS_PER_TILE,), jnp.int32),
          pltpu.VMEM((_SB,), jnp.float32),
          pltpu.VMEM((_SB,), jnp.float32),
          pltpu.VMEM((_SB,), jnp.int32),
          pltpu.VMEM((_NG, _G), jnp.int32),
          pltpu.VMEM((_G,), jnp.float32),
          pltpu.VMEM((_ZR, _C), jnp.float32),
          pltpu.VMEM((_SB,), jnp.float32),
          pltpu.VMEM((_MS, _C), jnp.float32),
          pltpu.VMEM((_MS,), jnp.float32),
          pltpu.VMEM((_MS + 16,), jnp.float32),
          pltpu.VMEM_SHARED((_ROWS, _C), jnp.float32),
          pltpu.VMEM_SHARED((_ROWS,), jnp.float32),
      ],
  )(feats_t, pim, ms)


def kernel(fv_features, points_img, proj_masks):
  feats_t = jnp.transpose(fv_features.reshape(_B, _C, _N), (0, 2, 1))
  pim = points_img[:, :2].reshape(_B, 2, _N)
  ms = proj_masks.reshape(_B, _N)
  mean = _sc_scatter(feats_t, pim, ms)
  return jnp.transpose(mean, (0, 2, 1)).reshape(_B, _C, _NY, _NX)


# speedup vs baseline: 1.3499x; 1.0640x over previous
import jax
import jax.numpy as jnp
from jax import lax
from jax.experimental import pallas as pl
from jax.experimental.pallas import tpu as pltpu
from jax.experimental.pallas import tpu_sc as plsc

_B, _C, _H, _W = 4, 32, 64, 2048
_N = _H * _W
_NX = _NY = 512
_NCELL = _NX * _NY
_NCHUNKS = 16
_CHUNK = _NCELL // _NCHUNKS
_DUMPS = 256
_ROWS = _CHUNK + _DUMPS

_NCORES = 2
_NSUB = 16
_PTS_PER_TILE = _N // _NSUB
_SB = 1024
_NSB = _PTS_PER_TILE // _SB
_G = 128
_NG = _SB // _G

_TROWS = _CHUNK // _NSUB
_YROWS = _TROWS // _NX
_ZR = 256
_MS = 512

_XY0 = -51.2
_VOX = 0.2


def _sc_body(feats_hbm, pim_hbm, ms_hbm, mean_hbm,
             feats_v, gidx_v, x_v, y_v, m_v, idx_v, ones_v, zrow_v, zcnt_v,
             mrow_v, mcnt_v, rcp_v, sums_sh, cnts_sh):
  cid = lax.axis_index("c")
  tid = lax.axis_index("s")
  lanes = jnp.arange(16, dtype=jnp.int32)
  ones16 = jnp.ones((16,), jnp.float32)
  zeros16 = jnp.zeros((16,), jnp.float32)

  for g in range(_G // 16):
    ones_v[pl.ds(g * 16, 16)] = ones16

  def _zr(i, c):
    zrow_v[i, pl.ds(0, 16)] = zeros16
    zrow_v[i, pl.ds(16, 16)] = zeros16
    return c
  lax.fori_loop(0, _ZR, _zr, 0)

  def _zc(i, c):
    zcnt_v[pl.ds(i * 16, 16)] = zeros16
    return c
  lax.fori_loop(0, _SB // 16, _zc, 0)

  def _batch(b, carry0):
    for sb in range(_NSB):
      pbase = tid * _PTS_PER_TILE + sb * _SB
      pltpu.sync_copy(pim_hbm.at[b, 0, pl.ds(pbase, _SB)], x_v)
      pltpu.sync_copy(pim_hbm.at[b, 1, pl.ds(pbase, _SB)], y_v)
      pltpu.sync_copy(ms_hbm.at[b, pl.ds(pbase, _SB)], m_v)

      def _ci(k, sb=sb):
        o = k * 16
        xx = x_v[pl.ds(o, 16)]
        yy = y_v[pl.ds(o, 16)]
        mm = m_v[pl.ds(o, 16)]
        cx = ((xx - _XY0) / _VOX).astype(jnp.int32)
        cx = jnp.minimum(jnp.maximum(cx, 0), _NX - 1)
        cy = ((yy - _XY0) / _VOX).astype(jnp.int32)
        cy = jnp.minimum(jnp.maximum(cy, 0), _NY - 1)
        flat = cy * _NX + cx
        flat = jnp.where(mm > 0, flat, _NCELL)
        gidx_v[pl.ds(sb * _SB + o, 16)] = flat
      plsc.parallel_loop(0, _SB // 16, 1, unroll=4)(_ci)

    def _pass(i, carry1):
      ch = i * _NCORES + cid
      cell0 = ch * _CHUNK

      for r in range(_TROWS // _ZR):
        pltpu.sync_copy(zrow_v, sums_sh.at[pl.ds(tid * _TROWS + r * _ZR, _ZR)])
      for r in range(_TROWS // _SB):
        pltpu.sync_copy(zcnt_v, cnts_sh.at[pl.ds(tid * _TROWS + r * _SB, _SB)])

      pltpu.sync_copy(zrow_v.at[pl.ds(0, 16)],
                      sums_sh.at[pl.ds(_CHUNK + tid * 16, 16)])
      pltpu.sync_copy(zcnt_v.at[pl.ds(0, 16)],
                      cnts_sh.at[pl.ds(_CHUNK + tid * 16, 16)])

      plsc.subcore_barrier()

      for sb in range(_NSB):
        pbase = tid * _PTS_PER_TILE + sb * _SB
        pltpu.sync_copy(feats_hbm.at[b, pl.ds(pbase, _SB)], feats_v)

        def _li(k, sb=sb):
          fl = gidx_v[pl.ds(sb * _SB + k * 16, 16)]
          loc = fl - cell0
          ok = (loc >= 0) & (loc < _CHUNK)
          idx_v[k // 8, pl.ds((k % 8) * 16, 16)] = jnp.where(
              ok, loc, _CHUNK + tid * 16 + lanes)
        plsc.parallel_loop(0, _SB // 16, 1, unroll=4)(_li)

        for g in range(_NG):
          pltpu.sync_copy(feats_v.at[pl.ds(g * _G, _G)],
                          sums_sh.at[idx_v.at[g]], add=True)
          pltpu.sync_copy(ones_v, cnts_sh.at[idx_v.at[g]], add=True)

      plsc.subcore_barrier()

      for s in range(_TROWS // _MS):
        row0 = tid * _TROWS + s * _MS
        pltpu.sync_copy(sums_sh.at[pl.ds(row0, _MS)], mrow_v)
        pltpu.sync_copy(cnts_sh.at[pl.ds(row0, _MS)], mcnt_v)

        def _rcp(k):
          cc = mcnt_v[pl.ds(k * 16, 16)]
          rcp_v[pl.ds(k * 16, 16)] = 1.0 / jnp.maximum(cc, 1.0)
        plsc.parallel_loop(0, _MS // 16, 1, unroll=2)(_rcp)

        def _mm(j):
          r = rcp_v[pl.ds(j, 16)][0]
          mrow_v[j, pl.ds(0, 16)] = mrow_v[j, pl.ds(0, 16)] * r
          mrow_v[j, pl.ds(16, 16)] = mrow_v[j, pl.ds(16, 16)] * r
        plsc.parallel_loop(0, _MS, 1, unroll=8)(_mm)

        pltpu.sync_copy(mrow_v,
                        mean_hbm.at[b, pl.ds(cell0 + row0, _MS)])

      plsc.subcore_barrier()
      return carry1

    lax.fori_loop(0, _NCHUNKS // _NCORES, _pass, 0)
    return carry0

  lax.fori_loop(0, _B, _batch, 0)


def _sc_scatter(feats_t, pim, ms):
  mesh = plsc.VectorSubcoreMesh(core_axis_name="c", subcore_axis_name="s",
                                num_cores=_NCORES, num_subcores=_NSUB)
  return pl.kernel(
      _sc_body,
      out_type=jax.ShapeDtypeStruct((_B, _NCELL, _C), jnp.float32),
      mesh=mesh,
      compiler_params=pltpu.CompilerParams(use_tc_tiling_on_sc=False,
                                           needs_layout_passes=False),
      scratch_types=[
          pltpu.VMEM((_SB, _C), jnp.float32),
          pltpu.VMEM((_PTS_PER_TILE,), jnp.int32),
          pltpu.VMEM((_SB,), jnp.float32),
          pltpu.VMEM((_SB,), jnp.float32),
          pltpu.VMEM((_SB,), jnp.int32),
          pltpu.VMEM((_NG, _G), jnp.int32),
          pltpu.VMEM((_G,), jnp.float32),
          pltpu.VMEM((_ZR, _C), jnp.float32),
          pltpu.VMEM((_SB,), jnp.float32),
          pltpu.VMEM((_MS, _C), jnp.float32),
          pltpu.VMEM((_MS,), jnp.float32),
          pltpu.VMEM((_MS + 16,), jnp.float32),
          pltpu.VMEM_SHARED((_ROWS, _C), jnp.float32),
          pltpu.VMEM_SHARED((_ROWS,), jnp.float32),
      ],
  )(feats_t, pim, ms)


def kernel(fv_features, points_img, proj_masks):
  feats_t = jnp.transpose(fv_features.reshape(_B, _C, _N), (0, 2, 1))
  pim = points_img[:, :2].reshape(_B, 2, _N)
  ms = proj_masks.reshape(_B, _N)
  mean = _sc_scatter(feats_t, pim, ms)
  return jnp.transpose(mean, (0, 2, 1)).reshape(_B, _C, _NY, _NX)


# speedup vs baseline: 1.8071x; 1.3388x over previous
import jax
import jax.numpy as jnp
from jax import lax
from jax.experimental import pallas as pl
from jax.experimental.pallas import tpu as pltpu
from jax.experimental.pallas import tpu_sc as plsc

_B, _C, _H, _W = 4, 32, 64, 2048
_N = _H * _W
_NX = _NY = 512
_NCELL = _NX * _NY
_NCHUNKS = 8
_CHUNK = _NCELL // _NCHUNKS
_DUMPS = 256
_ROWS = _CHUNK + _DUMPS

_NCORES = 2
_NSUB = 16
_PTS_PER_TILE = _N // _NSUB
_SB = 1024
_NSB = _PTS_PER_TILE // _SB
_G = 128
_NG = _SB // _G

_TROWS = _CHUNK // _NSUB
_YROWS = _TROWS // _NX
_ZR = 64
_MS = 256

_XY0 = -51.2
_VOX = 0.2


def _sc_body(feats_hbm, pim_hbm, ms_hbm, mean_hbm,
             feats_v, gidx_v, x_v, y_v, m_v, idx_v, ones_v, zrow_v, zcnt_v,
             mrow_v, mcnt_v, rcp_v, sums_sh, cnts_sh):
  cid = lax.axis_index("c")
  tid = lax.axis_index("s")
  lanes = jnp.arange(16, dtype=jnp.int32)
  ones16 = jnp.ones((16,), jnp.float32)
  zeros16 = jnp.zeros((16,), jnp.float32)

  for g in range(_G // 16):
    ones_v[pl.ds(g * 16, 16)] = ones16

  def _zr(i, c):
    zrow_v[i, pl.ds(0, 16)] = zeros16
    zrow_v[i, pl.ds(16, 16)] = zeros16
    return c
  lax.fori_loop(0, _ZR, _zr, 0)

  def _zc(i, c):
    zcnt_v[pl.ds(i * 16, 16)] = zeros16
    return c
  lax.fori_loop(0, _SB // 16, _zc, 0)

  def _batch(b, carry0):
    for sb in range(_NSB):
      pbase = tid * _PTS_PER_TILE + sb * _SB
      pltpu.sync_copy(pim_hbm.at[b, 0, pl.ds(pbase, _SB)], x_v)
      pltpu.sync_copy(pim_hbm.at[b, 1, pl.ds(pbase, _SB)], y_v)
      pltpu.sync_copy(ms_hbm.at[b, pl.ds(pbase, _SB)], m_v)

      def _ci(k, sb=sb):
        o = k * 16
        xx = x_v[pl.ds(o, 16)]
        yy = y_v[pl.ds(o, 16)]
        mm = m_v[pl.ds(o, 16)]
        cx = ((xx - _XY0) / _VOX).astype(jnp.int32)
        cx = jnp.minimum(jnp.maximum(cx, 0), _NX - 1)
        cy = ((yy - _XY0) / _VOX).astype(jnp.int32)
        cy = jnp.minimum(jnp.maximum(cy, 0), _NY - 1)
        flat = cy * _NX + cx
        flat = jnp.where(mm > 0, flat, _NCELL)
        gidx_v[pl.ds(sb * _SB + o, 16)] = flat
      plsc.parallel_loop(0, _SB // 16, 1, unroll=4)(_ci)

    def _pass(i, carry1):
      ch = i * _NCORES + cid
      cell0 = ch * _CHUNK

      for r in range(_TROWS // _ZR):
        pltpu.sync_copy(zrow_v, sums_sh.at[pl.ds(tid * _TROWS + r * _ZR, _ZR)])
      for r in range(_TROWS // _SB):
        pltpu.sync_copy(zcnt_v, cnts_sh.at[pl.ds(tid * _TROWS + r * _SB, _SB)])

      pltpu.sync_copy(zrow_v.at[pl.ds(0, 16)],
                      sums_sh.at[pl.ds(_CHUNK + tid * 16, 16)])
      pltpu.sync_copy(zcnt_v.at[pl.ds(0, 16)],
                      cnts_sh.at[pl.ds(_CHUNK + tid * 16, 16)])

      plsc.subcore_barrier()

      for sb in range(_NSB):
        pbase = tid * _PTS_PER_TILE + sb * _SB
        pltpu.sync_copy(feats_hbm.at[b, pl.ds(pbase, _SB)], feats_v)

        def _li(k, sb=sb):
          fl = gidx_v[pl.ds(sb * _SB + k * 16, 16)]
          loc = fl - cell0
          ok = (loc >= 0) & (loc < _CHUNK)
          idx_v[k // 8, pl.ds((k % 8) * 16, 16)] = jnp.where(
              ok, loc, _CHUNK + tid * 16 + lanes)
        plsc.parallel_loop(0, _SB // 16, 1, unroll=4)(_li)

        for g in range(_NG):
          pltpu.sync_copy(feats_v.at[pl.ds(g * _G, _G)],
                          sums_sh.at[idx_v.at[g]], add=True)
          pltpu.sync_copy(ones_v, cnts_sh.at[idx_v.at[g]], add=True)

      plsc.subcore_barrier()

      for s in range(_TROWS // _MS):
        row0 = tid * _TROWS + s * _MS
        pltpu.sync_copy(sums_sh.at[pl.ds(row0, _MS)], mrow_v)
        pltpu.sync_copy(cnts_sh.at[pl.ds(row0, _MS)], mcnt_v)

        def _rcp(k):
          cc = mcnt_v[pl.ds(k * 16, 16)]
          rcp_v[pl.ds(k * 16, 16)] = 1.0 / jnp.maximum(cc, 1.0)
        plsc.parallel_loop(0, _MS // 16, 1, unroll=2)(_rcp)

        def _mm(j):
          r = rcp_v[pl.ds(j, 16)][0]
          mrow_v[j, pl.ds(0, 16)] = mrow_v[j, pl.ds(0, 16)] * r
          mrow_v[j, pl.ds(16, 16)] = mrow_v[j, pl.ds(16, 16)] * r
        plsc.parallel_loop(0, _MS, 1, unroll=8)(_mm)

        pltpu.sync_copy(mrow_v,
                        mean_hbm.at[b, pl.ds(cell0 + row0, _MS)])

      plsc.subcore_barrier()
      return carry1

    lax.fori_loop(0, _NCHUNKS // _NCORES, _pass, 0)
    return carry0

  lax.fori_loop(0, _B, _batch, 0)


def _sc_scatter(feats_t, pim, ms):
  mesh = plsc.VectorSubcoreMesh(core_axis_name="c", subcore_axis_name="s",
                                num_cores=_NCORES, num_subcores=_NSUB)
  return pl.kernel(
      _sc_body,
      out_type=jax.ShapeDtypeStruct((_B, _NCELL, _C), jnp.float32),
      mesh=mesh,
      compiler_params=pltpu.CompilerParams(use_tc_tiling_on_sc=False,
                                           needs_layout_passes=False),
      scratch_types=[
          pltpu.VMEM((_SB, _C), jnp.float32),
          pltpu.VMEM((_PTS_PER_TILE,), jnp.int32),
          pltpu.VMEM((_SB,), jnp.float32),
          pltpu.VMEM((_SB,), jnp.float32),
          pltpu.VMEM((_SB,), jnp.int32),
          pltpu.VMEM((_NG, _G), jnp.int32),
          pltpu.VMEM((_G,), jnp.float32),
          pltpu.VMEM((_ZR, _C), jnp.float32),
          pltpu.VMEM((_SB,), jnp.float32),
          pltpu.VMEM((_MS, _C), jnp.float32),
          pltpu.VMEM((_MS,), jnp.float32),
          pltpu.VMEM((_MS + 16,), jnp.float32),
          pltpu.VMEM_SHARED((_ROWS, _C), jnp.float32),
          pltpu.VMEM_SHARED((_ROWS,), jnp.float32),
      ],
  )(feats_t, pim, ms)


def kernel(fv_features, points_img, proj_masks):
  feats_t = jnp.transpose(fv_features.reshape(_B, _C, _N), (0, 2, 1))
  pim = points_img[:, :2].reshape(_B, 2, _N)
  ms = proj_masks.reshape(_B, _N)
  mean = _sc_scatter(feats_t, pim, ms)
  return jnp.transpose(mean, (0, 2, 1)).reshape(_B, _C, _NY, _NX)
